# Initial kernel scaffold; baseline (speedup 1.0000x reference)
import jax
import jax.numpy as jnp
from jax.experimental import pallas as pl


def kernel(x_drug, x_disease, ei_treats, ei_treated_by, ei_has_contraind, ei_contraind_by, ei_has_parentcode, ei_has_childcode, ei_interacts, params):
    raise NotImplementedError("write your pallas kernel here")


# speedup vs baseline: 5.1877x; 5.1877x over previous
import functools

import jax
import jax.numpy as jnp
from jax import lax
from jax.experimental import pallas as pl
from jax.experimental.pallas import tpu as pltpu, tpu_sc as plsc

ND = 10000
NS = 50000
D = 128

RELS = ["treats", "has_contraind", "has_parentcode", "has_childcode",
        "treated_by", "contraind_by", "interacts"]
RSRC = ["d", "d", "s", "s", "s", "s", "d"]
RDST = ["s", "s", "s", "s", "d", "d", "d"]
RE = [160000, 80000, 25000, 25000, 160000, 80000, 80000]
RC = [0, 0, 0, 0, 1, 1, 1]

BSCAN = 2048
EALIGN = 16 * BSCAN
EPAD = [((e + EALIGN - 1) // EALIGN) * EALIGN for e in RE]
EOFF = [sum(EPAD[:i]) for i in range(7)]
EP_TOT = sum(EPAD)

CDIS = 6272
CDRG = 5120
NSP = 8 * CDIS
NDP = 2 * CDRG

_NSRC = {"d": ND, "s": NS}
_NDSTP = {"d": NDP, "s": NSP}
_NDST = {"d": ND, "s": NS}

AS_SZ = [_NSRC[t] for t in RSRC]
AD_SZ = [_NDSTP[t] for t in RDST]
AS_OFF = [sum(AS_SZ[:i]) for i in range(7)]
_AD0 = sum(AS_SZ)
AD_OFF = [_AD0 + sum(AD_SZ[:i]) for i in range(7)]
AV_TOT = _AD0 + sum(AD_SZ)

DOFF = [sum(AD_SZ[:i]) for i in range(7)]
DTOT = sum(AD_SZ)

HS_OFF = [0, ND, 0, 0, 0, NS, 2 * ND]
HS_OFF[2] = 2 * NS
HS_OFF[3] = 3 * NS

_mesh = plsc.VectorSubcoreMesh(core_axis_name="c", subcore_axis_name="s")


def _z16():
    return jnp.zeros((16,), jnp.float32)


def _pieces(n):
    out, o = [], 0
    while o < n:
        l = min(128, n - o)
        out.append((o, l))
        o += l
    return out


@functools.partial(
    pl.kernel,
    out_type=(jax.ShapeDtypeStruct((EP_TOT,), jnp.float32),
              jax.ShapeDtypeStruct((DTOT,), jnp.float32)),
    mesh=_mesh,
    compiler_params=pltpu.CompilerParams(needs_layout_passes=False),
    scratch_types=[
        pltpu.VMEM((50048,), jnp.float32),
        pltpu.VMEM((NSP,), jnp.float32),
        pltpu.VMEM((BSCAN,), jnp.int32),
        pltpu.VMEM((BSCAN,), jnp.int32),
        pltpu.VMEM((BSCAN,), jnp.float32),
        pltpu.VMEM((BSCAN,), jnp.int32),
        pltpu.VMEM((3136,), jnp.float32),
        pltpu.VMEM((3136,), jnp.float32),
        pltpu.VMEM((16,), jnp.float32),
        pltpu.VMEM_SHARED((DTOT + 64,), jnp.float32),
    ],
)
def _p1(src_all, dst_all, av_all, mvec, ex_all, den_all,
        as_tab, ad_tab, srcb, dstb, exb, didx, zbuf, bnc, mv, den_sh):
    cid = lax.axis_index("c")
    sid = lax.axis_index("s")
    pltpu.sync_copy(mvec, mv)
    lane = lax.iota(jnp.int32, 16)

    def zb(i, c):
        zbuf[pl.ds(i * 16, 16)] = _z16()
        return c
    lax.fori_loop(0, 3136 // 16, zb, 0)

    for r in range(7):
        ns = _NSRC[RSRC[r]]
        ndst = _NDST[RDST[r]]
        ndp = _NDSTP[RDST[r]]
        per = ndp // 16
        S = EPAD[r] // 16
        nblk = S // BSCAN

        @pl.when(cid == RC[r])
        def _run(r=r, ns=ns, ndst=ndst, ndp=ndp, per=per, S=S, nblk=nblk):
            pltpu.sync_copy(zbuf.at[pl.ds(0, per)],
                            den_sh.at[pl.ds(DOFF[r] + sid * per, per)])
            plsc.subcore_barrier()
            pltpu.sync_copy(av_all.at[pl.ds(AS_OFF[r], ns)],
                            as_tab.at[pl.ds(0, ns)])
            pltpu.sync_copy(av_all.at[pl.ds(AD_OFF[r], ndp)],
                            ad_tab.at[pl.ds(0, ndp)])
            mr = mv[...][r]
            base_r = EOFF[r] + sid * S

            def blk(b, cb, r=r, ndst=ndst, mr=mr, base_r=base_r):
                off = base_r + b * BSCAN
                pltpu.sync_copy(src_all.at[pl.ds(off, BSCAN)], srcb)
                pltpu.sync_copy(dst_all.at[pl.ds(off, BSCAN)], dstb)

                def body(jr, c):
                    for q in range(8):
                        sl = pl.ds(jr * 128 + q * 16, 16)
                        s16 = srcb[sl]
                        d16 = dstb[sl]
                        ok = d16 < ndst
                        a1 = plsc.load_gather(as_tab, [s16])
                        a2 = plsc.load_gather(ad_tab, [jnp.where(ok, d16, 0)])
                        z = a1 + a2
                        e = jnp.where(z >= 0.0, z, 0.2 * z)
                        ex = jnp.where(ok, jnp.exp(e - mr), 0.0)
                        exb[sl] = ex
                        dmp = (lane + (jr * 128 + q * 16)) & 63
                        didx[sl] = DOFF[r] + jnp.where(ok, d16, dmp)
                    return c
                lax.fori_loop(0, 16, body, 0)
                pltpu.sync_copy(exb, ex_all.at[pl.ds(off, BSCAN)])

                def sc(jj, c):
                    pltpu.sync_copy(
                        exb.at[pl.ds(jj * 128, 128)],
                        den_sh.at[didx.at[pl.ds(jj * 128, 128)]], add=True)
                    return c
                lax.fori_loop(0, 16, sc, 0)
                return cb
            lax.fori_loop(0, nblk, blk, 0)
            plsc.subcore_barrier()
            pltpu.sync_copy(den_sh.at[pl.ds(DOFF[r] + sid * per, per)],
                            bnc.at[pl.ds(0, per)])
            pltpu.sync_copy(bnc.at[pl.ds(0, per)],
                            den_all.at[pl.ds(DOFF[r] + sid * per, per)])


def _make_p2(rels, C, half):
    hrows = 2 * C
    zrows = C + 64
    zpt = -(-zrows // 16)
    zr_tot = 16 * zpt
    wpt = C // 16

    @functools.partial(
        pl.kernel,
        out_type=jax.ShapeDtypeStruct((hrows, D), jnp.float32),
        mesh=_mesh,
        compiler_params=pltpu.CompilerParams(needs_layout_passes=False),
        scratch_types=[
            pltpu.VMEM((C,), jnp.float32),
            pltpu.VMEM((BSCAN,), jnp.int32),
            pltpu.VMEM((BSCAN,), jnp.int32),
            pltpu.VMEM((BSCAN,), jnp.float32),
            pltpu.VMEM((BSCAN,), jnp.float32),
            pltpu.VMEM((BSCAN,), jnp.int32),
            pltpu.VMEM((BSCAN,), jnp.int32),
            pltpu.VMEM((128, D), jnp.float32),
            pltpu.VMEM_SHARED((zr_tot, D), jnp.float32),
            pltpu.SemaphoreType.DMA,
        ],
    )
    def _p2(src_all, dst_all, ex_all, den_all, hs_d, hs_s, out_h,
            den_loc, srcb, dstb, exb, wvb, gsrcb, didx, rows, out_sh, sem):
        cid = lax.axis_index("c")
        sid = lax.axis_index("s")
        lane = lax.iota(jnp.int32, 16)

        def zr(i, c):
            rows[i // 8, pl.ds((i % 8) * 16, 16)] = _z16()
            return c
        lax.fori_loop(0, 1024, zr, 0)
        zbase = sid * zpt
        for (o, l) in _pieces(zpt):
            pltpu.sync_copy(rows.at[pl.ds(0, l)],
                            out_sh.at[pl.ds(zbase + o, l)])
        plsc.subcore_barrier()

        if half is None:
            chunk_lo = cid * C
        else:
            chunk_lo = (2 * half + cid) * C
        loc0 = cid * 0

        for r in rels:
            hs_tab = hs_d if RSRC[r] == "d" else hs_s
            S = EPAD[r] // 16
            nblk = S // BSCAN
            pltpu.sync_copy(den_all.at[pl.ds(DOFF[r] + chunk_lo, C)], den_loc)
            base_r = EOFF[r] + sid * S

            def blk(b, cb, r=r, base_r=base_r):
                off = base_r + b * BSCAN
                pltpu.sync_copy(src_all.at[pl.ds(off, BSCAN)], srcb)
                pltpu.sync_copy(dst_all.at[pl.ds(off, BSCAN)], dstb)
                pltpu.sync_copy(ex_all.at[pl.ds(off, BSCAN)], exb)

                def body(jr, c, r=r):
                    for q in range(8):
                        sl = pl.ds(jr * 128 + q * 16, 16)
                        s16 = srcb[sl]
                        d16 = dstb[sl]
                        ex = exb[sl]
                        dl = d16 - chunk_lo
                        ok = (dl >= 0) & (dl < C)
                        den = plsc.load_gather(
                            den_loc, [jnp.where(ok, dl, 0)])
                        w = ex / jnp.maximum(den, 1e-16)
                        wvb[sl] = jnp.where(ok, w, 0.0)
                        gsrcb[sl] = s16 + HS_OFF[r]
                        dmp = C + ((lane + (jr * 128 + q * 16)) & 63)
                        didx[sl] = jnp.where(ok, dl, dmp)
                    return c
                lax.fori_loop(0, 16, body, 0)

                def jloop(jj, c):
                    pltpu.async_copy(
                        hs_tab.at[gsrcb.at[pl.ds(jj * 128, 128)]],
                        rows, sem).wait()

                    def sb(jv, c2):
                        w16 = wvb[pl.ds(jj * 128 + jv * 16, 16)]
                        for l in range(16):
                            cs = w16[l]
                            ei = jv * 16 + l
                            for q in range(8):
                                qs = pl.ds(q * 16, 16)
                                rows[ei, qs] = rows[ei, qs] * cs
                        return c2
                    lax.fori_loop(0, 8, sb, 0)
                    pltpu.sync_copy(
                        rows, out_sh.at[didx.at[pl.ds(jj * 128, 128)]],
                        add=True)
                    return c
                lax.fori_loop(0, 16, jloop, 0)
                return cb
            lax.fori_loop(0, nblk, blk, 0)
        plsc.subcore_barrier()
        wbase = sid * wpt + loc0
        for (o, l) in _pieces(wpt):
            pltpu.sync_copy(out_sh.at[pl.ds(wbase + o, l)],
                            rows.at[pl.ds(0, l)])
            pltpu.sync_copy(rows.at[pl.ds(0, l)],
                            out_h.at[pl.ds(cid * C + wbase + o, l)])

    return _p2


_p2_dis = [_make_p2([0, 1, 2, 3], CDIS, q) for q in range(4)]
_p2_drug = _make_p2([4, 5, 6], CDRG, None)


def _mm_hs(x, w, bias=None):
    G = w.shape[0]
    N = x.shape[0]
    TN = 1000
    nt = N // TN
    has_b = bias is not None

    def kern(*refs):
        if has_b:
            x_ref, w_ref, b_ref, o_ref = refs
            xa = jnp.maximum(x_ref[...] + b_ref[...], 0.0)
        else:
            x_ref, w_ref, o_ref = refs
            xa = x_ref[...]
        o_ref[0] = jnp.dot(xa, w_ref[0], preferred_element_type=jnp.float32)

    in_specs = [pl.BlockSpec((TN, D), lambda g, t: (t, 0)),
                pl.BlockSpec((1, D, D), lambda g, t: (g, 0, 0))]
    args = [x, w]
    if has_b:
        in_specs.append(pl.BlockSpec((1, D), lambda g, t: (0, 0)))
        args.append(bias.reshape(1, D))
    return pl.pallas_call(
        kern, grid=(G, nt), in_specs=in_specs,
        out_specs=pl.BlockSpec((1, TN, D), lambda g, t: (g, t, 0)),
        out_shape=jax.ShapeDtypeStruct((G, N, D), jnp.float32),
    )(*args)


def _mm_acol(x, w, bias=None):
    N = x.shape[0]
    TN = 1000
    nt = N // TN
    has_b = bias is not None

    def kern(*refs):
        if has_b:
            x_ref, w_ref, b_ref, o_ref, cm_ref = refs
            xa = jnp.maximum(x_ref[...] + b_ref[...], 0.0)
        else:
            x_ref, w_ref, o_ref, cm_ref = refs
            xa = x_ref[...]
        y = jnp.dot(xa, w_ref[...], preferred_element_type=jnp.float32)
        o_ref[...] = y
        mb = jnp.broadcast_to(jnp.max(y, axis=0, keepdims=True), (8, D))

        @pl.when(pl.program_id(0) == 0)
        def _():
            cm_ref[...] = mb

        @pl.when(pl.program_id(0) > 0)
        def _():
            cm_ref[...] = jnp.maximum(cm_ref[...], mb)

    in_specs = [pl.BlockSpec((TN, D), lambda t: (t, 0)),
                pl.BlockSpec((D, D), lambda t: (0, 0))]
    args = [x, w]
    if has_b:
        in_specs.append(pl.BlockSpec((1, D), lambda t: (0, 0)))
        args.append(bias.reshape(1, D))
    return pl.pallas_call(
        kern, grid=(nt,), in_specs=in_specs,
        out_specs=(pl.BlockSpec((TN, D), lambda t: (t, 0)),
                   pl.BlockSpec((8, D), lambda t: (0, 0))),
        out_shape=(jax.ShapeDtypeStruct((N, D), jnp.float32),
                   jax.ShapeDtypeStruct((8, D), jnp.float32)),
    )(*args)


def _mm_lin(x, w, bias):
    N = x.shape[0]
    TN = 1000
    nt = N // TN

    def kern(x_ref, w_ref, b_ref, o_ref):
        y = jnp.dot(x_ref[...], w_ref[...], preferred_element_type=jnp.float32)
        o_ref[...] = jnp.maximum(y + b_ref[...], 0.0)

    return pl.pallas_call(
        kern, grid=(nt,),
        in_specs=[pl.BlockSpec((TN, D), lambda t: (t, 0)),
                  pl.BlockSpec((D, D), lambda t: (0, 0)),
                  pl.BlockSpec((1, D), lambda t: (0, 0))],
        out_specs=pl.BlockSpec((TN, D), lambda t: (t, 0)),
        out_shape=jax.ShapeDtypeStruct((N, D), jnp.float32),
    )(x, w, bias.reshape(1, D))


def _lrelu(z):
    return jnp.where(z >= 0.0, z, 0.2 * z)


def _pad1(v, n):
    return jnp.concatenate([v, jnp.zeros((n - v.shape[0],), v.dtype)])


def _layer_sc(src_all, dst_all, lp, x_d, x_s, bias_d=None, bias_s=None):
    w_d = jnp.stack([lp["treats"]["W_src"], lp["has_contraind"]["W_src"],
                     lp["interacts"]["W_src"]])
    w_s = jnp.stack([lp["treated_by"]["W_src"], lp["contraind_by"]["W_src"],
                     lp["has_parentcode"]["W_src"], lp["has_childcode"]["W_src"]])
    acd = jnp.stack([
        lp["treats"]["W_src"] @ lp["treats"]["att_src"],
        lp["has_contraind"]["W_src"] @ lp["has_contraind"]["att_src"],
        lp["interacts"]["W_src"] @ lp["interacts"]["att_src"],
        lp["treated_by"]["W_dst"] @ lp["treated_by"]["att_dst"],
        lp["contraind_by"]["W_dst"] @ lp["contraind_by"]["att_dst"],
        lp["interacts"]["W_dst"] @ lp["interacts"]["att_dst"],
    ], axis=1)
    acs = jnp.stack([
        lp["treated_by"]["W_src"] @ lp["treated_by"]["att_src"],
        lp["contraind_by"]["W_src"] @ lp["contraind_by"]["att_src"],
        lp["has_parentcode"]["W_src"] @ lp["has_parentcode"]["att_src"],
        lp["has_childcode"]["W_src"] @ lp["has_childcode"]["att_src"],
        lp["treats"]["W_dst"] @ lp["treats"]["att_dst"],
        lp["has_contraind"]["W_dst"] @ lp["has_contraind"]["att_dst"],
        lp["has_parentcode"]["W_dst"] @ lp["has_parentcode"]["att_dst"],
        lp["has_childcode"]["W_dst"] @ lp["has_childcode"]["att_dst"],
    ], axis=1)
    acd = jnp.pad(acd, ((0, 0), (0, D - acd.shape[1])))
    acs = jnp.pad(acs, ((0, 0), (0, D - acs.shape[1])))

    yd = _mm_hs(x_d, w_d, bias_d)
    ys = _mm_hs(x_s, w_s, bias_s)
    ad_col, cmd8 = _mm_acol(x_d, acd, bias_d)
    as_col, cms8 = _mm_acol(x_s, acs, bias_s)
    cmd = cmd8[0]
    cms = cms8[0]

    a_s = [ad_col[:, 0], ad_col[:, 1], as_col[:, 2], as_col[:, 3],
           as_col[:, 0], as_col[:, 1], ad_col[:, 2]]
    a_d = [as_col[:, 4], as_col[:, 5], as_col[:, 6], as_col[:, 7],
           ad_col[:, 3], ad_col[:, 4], ad_col[:, 5]]
    av = jnp.concatenate(
        [a_s[r] for r in range(7)]
        + [_pad1(a_d[r], AD_SZ[r]) for r in range(7)])

    mM = [
        _lrelu(cmd[0] + cms[4]), _lrelu(cmd[1] + cms[5]),
        _lrelu(cms[2] + cms[6]), _lrelu(cms[3] + cms[7]),
        _lrelu(cms[0] + cmd[3]), _lrelu(cms[1] + cmd[4]),
        _lrelu(cmd[2] + cmd[5]),
    ]
    mvec = _pad1(jnp.stack(mM), 16)

    hs_d = yd.reshape(3 * ND, D)
    hs_s = ys.reshape(4 * NS, D)

    ex_all, den_all = _p1(src_all, dst_all, av, mvec)
    quarters = [p2(src_all, dst_all, ex_all, den_all, hs_d, hs_s)
                for p2 in _p2_dis]
    drug = _p2_drug(src_all, dst_all, ex_all, den_all, hs_d, hs_s)
    dis = jnp.concatenate(quarters, axis=0)
    return drug[:ND], dis[:NS]


def kernel(x_drug, x_disease, ei_treats, ei_treated_by, ei_has_contraind,
           ei_contraind_by, ei_has_parentcode, ei_has_childcode,
           ei_interacts, params):
    eis = [ei_treats, ei_has_contraind, ei_has_parentcode, ei_has_childcode,
           ei_treated_by, ei_contraind_by, ei_interacts]
    srcs, dsts = [], []
    for r in range(7):
        e = RE[r]
        padn = EPAD[r] - e
        spread = jnp.arange(padn, dtype=jnp.int32) % _NSRC[RSRC[r]]
        srcs.append(jnp.concatenate([eis[r][0], spread]))
        dsts.append(jnp.concatenate(
            [eis[r][1], jnp.full((padn,), 1 << 30, jnp.int32)]))
    src_all = jnp.concatenate(srcs)
    dst_all = jnp.concatenate(dsts)

    b_dis1 = sum(params["l1"][RELS[r]]["b"] for r in range(4))
    b_drug1 = sum(params["l1"][RELS[r]]["b"] for r in range(4, 7))
    b_dis2 = sum(params["l2"][RELS[r]]["b"] for r in range(4))
    b_drug2 = sum(params["l2"][RELS[r]]["b"] for r in range(4, 7))

    d1, s1 = _layer_sc(src_all, dst_all, params["l1"], x_drug, x_disease)
    d2, s2 = _layer_sc(src_all, dst_all, params["l2"], d1, s1,
                       bias_d=b_drug1, bias_s=b_dis1)

    wld = params["lin_drug"]["W"]
    wls = params["lin_disease"]["W"]
    bld = b_drug2 @ wld + params["lin_drug"]["b"]
    bls = b_dis2 @ wls + params["lin_disease"]["b"]
    drug_out = _mm_lin(d2, wld, bld)
    dis_out = _mm_lin(s2, wls, bls)
    return drug_out, dis_out
